# baseline (speedup 1.0000x reference)
import functools

import jax
import jax.numpy as jnp
from jax import lax
from jax.experimental import pallas as pl
from jax.experimental.pallas import tpu as pltpu
from jax.experimental.pallas import tpu_sc as plsc

NC = 2
NS = 16
NW = NC * NS
CHUNK = 128
SLOPE = (1.0 / 8.0 + 1.0 / 3.0) / 2.0


def _mesh():
    return plsc.VectorSubcoreMesh(core_axis_name="c", subcore_axis_name="s")


def _zero_2d(ref, nrows, ncols):
    z = jnp.zeros((16,), jnp.float32)

    def body(r, _):
        for j in range(ncols // 16):
            ref[r, pl.ds(j * 16, 16)] = z
        return 0

    lax.fori_loop(0, nrows, body, 0)


@functools.partial(jax.jit, static_argnames=("npad",))
def _segsum(table, gidx_all, sidx_all, npad):
    epad = gidx_all.shape[0]
    e_per_tile = epad // NW
    n_chunks = e_per_tile // CHUNK
    rpt = npad // NS
    d = table.shape[1]

    @functools.partial(
        pl.kernel,
        mesh=_mesh(),
        out_type=jax.ShapeDtypeStruct((NC * npad, d), jnp.float32),
        scratch_types=[
            pltpu.VMEM((CHUNK,), jnp.int32),
            pltpu.VMEM((CHUNK,), jnp.int32),
            pltpu.VMEM((CHUNK, d), jnp.float32),
            pltpu.VMEM_SHARED((npad, d), jnp.float32),
            pltpu.SemaphoreType.DMA,
        ],
    )
    def k(table_hbm, g_hbm, s_hbm, out_hbm, gidx, sidx, rows, acc, sem):
        cid = lax.axis_index("c")
        sid = lax.axis_index("s")
        wid = sid * NC + cid
        n_acc = rpt // CHUNK
        _zero_2d(rows, CHUNK, d)

        def zbody(c, _):
            pltpu.sync_copy(rows, acc.at[pl.ds(sid * rpt + c * CHUNK, CHUNK)])
            return 0

        lax.fori_loop(0, n_acc, zbody, 0)
        plsc.subcore_barrier()
        base = wid * e_per_tile

        def body(c, _):
            off = base + c * CHUNK
            pltpu.sync_copy(g_hbm.at[pl.ds(off, CHUNK)], gidx)
            pltpu.sync_copy(s_hbm.at[pl.ds(off, CHUNK)], sidx)
            pltpu.async_copy(table_hbm.at[gidx], rows, sem).wait()
            pltpu.sync_copy(rows, acc.at[sidx], add=True)
            return 0

        lax.fori_loop(0, n_chunks, body, 0)
        plsc.subcore_barrier()

        def obody(c, _):
            r0 = sid * rpt + c * CHUNK
            pltpu.sync_copy(acc.at[pl.ds(r0, CHUNK)], rows)
            pltpu.sync_copy(rows, out_hbm.at[pl.ds(cid * npad + r0, CHUNK)])
            return 0

        lax.fori_loop(0, n_acc, obody, 0)

    return k(table, gidx_all, sidx_all)


@jax.jit
def _gather(table, idx):
    b = idx.shape[0]
    per_tile = b // NW
    gchunk = 64
    n_chunks = per_tile // gchunk
    d = table.shape[1]

    @functools.partial(
        pl.kernel,
        mesh=_mesh(),
        out_type=jax.ShapeDtypeStruct((b, d), jnp.float32),
        scratch_types=[
            pltpu.VMEM((gchunk,), jnp.int32),
            pltpu.VMEM((gchunk, d), jnp.float32),
            pltpu.SemaphoreType.DMA,
        ],
    )
    def k(table_hbm, idx_hbm, out_hbm, gidx, rows, sem):
        cid = lax.axis_index("c")
        sid = lax.axis_index("s")
        wid = sid * NC + cid
        base = wid * per_tile

        def body(c, _):
            off = base + c * gchunk
            pltpu.sync_copy(idx_hbm.at[pl.ds(off, gchunk)], gidx)
            pltpu.async_copy(table_hbm.at[gidx], rows, sem).wait()
            pltpu.sync_copy(rows, out_hbm.at[pl.ds(off, gchunk)])
            return 0

        lax.fori_loop(0, n_chunks, body, 0)

    return k(table, idx)


def _layer(aggx, aggr, deg, x, w_n, w_loop, n, npad):
    d = x.shape[1]
    nb = 512
    grid = npad // nb

    def body(ax_ref, ar_ref, dg_ref, x_ref, wn_ref, wl_ref, o_ref):
        i = pl.program_id(0)
        ax = ax_ref[0] + ax_ref[1]
        ar = ar_ref[0] + ar_ref[1]
        dg = dg_ref[0, :, 0:1] + dg_ref[1, :, 0:1]
        invd = 1.0 / jnp.maximum(dg, 1.0)
        pre = (ax - ar) * invd
        h = jnp.dot(pre, wn_ref[...], preferred_element_type=jnp.float32)
        h = h + jnp.dot(x_ref[...], wl_ref[...], preferred_element_type=jnp.float32)
        h = jnp.where(h >= 0.0, h, SLOPE * h)
        row = i * nb + lax.broadcasted_iota(jnp.int32, (nb, 1), 0)
        o_ref[...] = jnp.where(row < n, h, 0.0)

    return pl.pallas_call(
        body,
        grid=(grid,),
        in_specs=[
            pl.BlockSpec((2, nb, d), lambda i: (0, i, 0)),
            pl.BlockSpec((2, nb, d), lambda i: (0, i, 0)),
            pl.BlockSpec((2, nb, d), lambda i: (0, i, 0)),
            pl.BlockSpec((nb, d), lambda i: (i, 0)),
            pl.BlockSpec((d, d), lambda i: (0, 0)),
            pl.BlockSpec((d, d), lambda i: (0, 0)),
        ],
        out_specs=pl.BlockSpec((nb, d), lambda i: (i, 0)),
        out_shape=jax.ShapeDtypeStruct((npad, d), jnp.float32),
    )(aggx, aggr, deg, x, w_n, w_loop)


def _time(intervals, lens, pos, t_w, t_b):
    bl = intervals.shape[0]
    t = t_w.shape[1]

    def body(iv_ref, len_ref, pos_ref, tw_ref, tb_ref, o_ref):
        tm = jnp.where(pos_ref[...] < len_ref[...], iv_ref[...], 1000000)
        tmf = tm.astype(jnp.float32)
        arg = tmf * tw_ref[...] + tb_ref[...]
        o_ref[...] = jnp.cos(arg)

    return pl.pallas_call(
        body,
        out_shape=jax.ShapeDtypeStruct((bl, t), jnp.float32),
    )(intervals, lens, pos, t_w, t_b)


def kernel(s, ent_embeds, rel_embeds, day_embeds, t_w, t_b, s_len_non_zero,
           s_time_iterval_sorted, s_time_day, s_time_week, node_ids_graph,
           edge_index, edge_type, W_n1, W_loop1, W_n2, W_loop2):
    n, d = ent_embeds.shape
    e = edge_index.shape[1]
    b, l = node_ids_graph.shape
    npad = ((n + 2047) // 2048) * 2048
    epad = ((e + NW * CHUNK - 1) // (NW * CHUNK)) * (NW * CHUNK)

    src = edge_index[0].astype(jnp.int32)
    dst = edge_index[1].astype(jnp.int32)
    et = edge_type.astype(jnp.int32)
    pe = epad - e
    src_p = jnp.concatenate([src, jnp.zeros((pe,), jnp.int32)])
    dst_p = jnp.concatenate([dst, jnp.full((pe,), npad - 1, jnp.int32)])
    et_p = jnp.concatenate([et, jnp.zeros((pe,), jnp.int32)])
    x_pad = jnp.concatenate([ent_embeds, jnp.zeros((npad - n, d), jnp.float32)])

    aggr = _segsum(rel_embeds, et_p, dst_p, npad).reshape(NC, npad, d)
    ones_tab = jnp.ones((8, d), jnp.float32)
    zidx = jnp.zeros((epad,), jnp.int32)
    deg = _segsum(ones_tab, zidx, dst_p, npad).reshape(NC, npad, d)

    aggx1 = _segsum(x_pad, src_p, dst_p, npad).reshape(NC, npad, d)
    h1 = _layer(aggx1, aggr, deg, x_pad, W_n1, W_loop1, n, npad)
    aggx2 = _segsum(h1, src_p, dst_p, npad).reshape(NC, npad, d)
    h2 = _layer(aggx2, aggr, deg, h1, W_n2, W_loop2, n, npad)

    nid = node_ids_graph.reshape(-1).astype(jnp.int32)
    emb = _gather(h2, nid).reshape(b, l, d)

    iv = s_time_iterval_sorted.reshape(b * l, 1).astype(jnp.int32)
    lens = jnp.repeat(s_len_non_zero.astype(jnp.int32), l).reshape(b * l, 1)
    pos = jnp.tile(jnp.arange(l, dtype=jnp.int32), b).reshape(b * l, 1)
    te = _time(iv, lens, pos, t_w, t_b).reshape(b, l, -1)

    return jnp.concatenate([emb, te], axis=-1)


# speedup vs baseline: 9.3961x; 9.3961x over previous
import functools

import jax
import jax.numpy as jnp
from jax import lax
from jax.experimental import pallas as pl
from jax.experimental.pallas import tpu as pltpu
from jax.experimental.pallas import tpu_sc as plsc

NC = 2
NS = 16
NW = NC * NS
CHUNK = 128
SLOPE = (1.0 / 8.0 + 1.0 / 3.0) / 2.0


def _mesh():
    return plsc.VectorSubcoreMesh(core_axis_name="c", subcore_axis_name="s")


def _zero_2d(ref, nrows, ncols):
    z = jnp.zeros((16,), jnp.float32)

    def body(r, _):
        for j in range(ncols // 16):
            ref[r, pl.ds(j * 16, 16)] = z
        return 0

    lax.fori_loop(0, nrows, body, 0)


@functools.partial(jax.jit, static_argnames=("npad", "ones_rows"))
def _segsum(table, gidx_all, sidx_all, npad, ones_rows=False):
    epad = gidx_all.shape[0]
    e_per_tile = epad // NW
    n_chunks = e_per_tile // CHUNK
    rpt = npad // NS
    d = table.shape[1]

    @functools.partial(
        pl.kernel,
        mesh=_mesh(),
        out_type=jax.ShapeDtypeStruct((NC * npad, d), jnp.float32),
        scratch_types=[
            pltpu.VMEM((CHUNK,), jnp.int32),
            pltpu.VMEM((CHUNK,), jnp.int32),
            pltpu.VMEM((CHUNK, d), jnp.float32),
            pltpu.VMEM_SHARED((npad, d), jnp.float32),
            pltpu.SemaphoreType.DMA,
        ],
    )
    def k(table_hbm, g_hbm, s_hbm, out_hbm, gidx, sidx, rows, acc, sem):
        cid = lax.axis_index("c")
        sid = lax.axis_index("s")
        wid = sid * NC + cid
        n_acc = rpt // CHUNK
        _zero_2d(rows, CHUNK, d)

        def zbody(c, _):
            pltpu.sync_copy(rows, acc.at[pl.ds(sid * rpt + c * CHUNK, CHUNK)])
            return 0

        lax.fori_loop(0, n_acc, zbody, 0)
        plsc.subcore_barrier()
        base = wid * e_per_tile
        if ones_rows:
            one16 = jnp.ones((16,), jnp.float32)

            def fbody(r, _):
                for j in range(d // 16):
                    rows[r, pl.ds(j * 16, 16)] = one16
                return 0

            lax.fori_loop(0, CHUNK, fbody, 0)

        def body(c, _):
            off = base + c * CHUNK
            pltpu.sync_copy(s_hbm.at[pl.ds(off, CHUNK)], sidx)
            if not ones_rows:
                pltpu.sync_copy(g_hbm.at[pl.ds(off, CHUNK)], gidx)
                pltpu.async_copy(table_hbm.at[gidx], rows, sem).wait()
            pltpu.sync_copy(rows, acc.at[sidx], add=True)
            return 0

        lax.fori_loop(0, n_chunks, body, 0)
        plsc.subcore_barrier()

        def obody(c, _):
            r0 = sid * rpt + c * CHUNK
            pltpu.sync_copy(acc.at[pl.ds(r0, CHUNK)], rows)
            pltpu.sync_copy(rows, out_hbm.at[pl.ds(cid * npad + r0, CHUNK)])
            return 0

        lax.fori_loop(0, n_acc, obody, 0)

    return k(table, gidx_all, sidx_all)


@jax.jit
def _gather(table, idx):
    b = idx.shape[0]
    per_tile = b // NW
    gchunk = 64
    n_chunks = per_tile // gchunk
    d = table.shape[1]

    @functools.partial(
        pl.kernel,
        mesh=_mesh(),
        out_type=jax.ShapeDtypeStruct((b, d), jnp.float32),
        scratch_types=[
            pltpu.VMEM((gchunk,), jnp.int32),
            pltpu.VMEM((gchunk, d), jnp.float32),
            pltpu.SemaphoreType.DMA,
        ],
    )
    def k(table_hbm, idx_hbm, out_hbm, gidx, rows, sem):
        cid = lax.axis_index("c")
        sid = lax.axis_index("s")
        wid = sid * NC + cid
        base = wid * per_tile

        def body(c, _):
            off = base + c * gchunk
            pltpu.sync_copy(idx_hbm.at[pl.ds(off, gchunk)], gidx)
            pltpu.async_copy(table_hbm.at[gidx], rows, sem).wait()
            pltpu.sync_copy(rows, out_hbm.at[pl.ds(off, gchunk)])
            return 0

        lax.fori_loop(0, n_chunks, body, 0)

    return k(table, idx)


def _layer(aggx, aggr, deg, x, w_n, w_loop, n, npad):
    d = x.shape[1]
    nb = 512
    grid = npad // nb

    def body(ax_ref, ar_ref, dg_ref, x_ref, wn_ref, wl_ref, o_ref):
        i = pl.program_id(0)
        ax = ax_ref[0] + ax_ref[1]
        ar = ar_ref[0] + ar_ref[1]
        dg = dg_ref[0, :, 0:1] + dg_ref[1, :, 0:1]
        invd = 1.0 / jnp.maximum(dg, 1.0)
        pre = (ax - ar) * invd
        h = jnp.dot(pre, wn_ref[...], preferred_element_type=jnp.float32)
        h = h + jnp.dot(x_ref[...], wl_ref[...], preferred_element_type=jnp.float32)
        h = jnp.where(h >= 0.0, h, SLOPE * h)
        row = i * nb + lax.broadcasted_iota(jnp.int32, (nb, 1), 0)
        o_ref[...] = jnp.where(row < n, h, 0.0)

    return pl.pallas_call(
        body,
        grid=(grid,),
        in_specs=[
            pl.BlockSpec((2, nb, d), lambda i: (0, i, 0)),
            pl.BlockSpec((2, nb, d), lambda i: (0, i, 0)),
            pl.BlockSpec((2, nb, d), lambda i: (0, i, 0)),
            pl.BlockSpec((nb, d), lambda i: (i, 0)),
            pl.BlockSpec((d, d), lambda i: (0, 0)),
            pl.BlockSpec((d, d), lambda i: (0, 0)),
        ],
        out_specs=pl.BlockSpec((nb, d), lambda i: (i, 0)),
        out_shape=jax.ShapeDtypeStruct((npad, d), jnp.float32),
    )(aggx, aggr, deg, x, w_n, w_loop)


def _time(intervals, lens, pos, t_w, t_b):
    bl = intervals.shape[0]
    t = t_w.shape[1]

    def body(iv_ref, len_ref, pos_ref, tw_ref, tb_ref, o_ref):
        tm = jnp.where(pos_ref[...] < len_ref[...], iv_ref[...], 1000000)
        tmf = tm.astype(jnp.float32)
        arg = tmf * tw_ref[...] + tb_ref[...]
        o_ref[...] = jnp.cos(arg)

    return pl.pallas_call(
        body,
        out_shape=jax.ShapeDtypeStruct((bl, t), jnp.float32),
    )(intervals, lens, pos, t_w, t_b)


def kernel(s, ent_embeds, rel_embeds, day_embeds, t_w, t_b, s_len_non_zero,
           s_time_iterval_sorted, s_time_day, s_time_week, node_ids_graph,
           edge_index, edge_type, W_n1, W_loop1, W_n2, W_loop2):
    n, d = ent_embeds.shape
    e = edge_index.shape[1]
    b, l = node_ids_graph.shape
    npad = ((n + 2047) // 2048) * 2048
    epad = ((e + NW * CHUNK - 1) // (NW * CHUNK)) * (NW * CHUNK)

    src = edge_index[0].astype(jnp.int32)
    dst = edge_index[1].astype(jnp.int32)
    et = edge_type.astype(jnp.int32)
    pe = epad - e
    src_p = jnp.concatenate([src, jnp.zeros((pe,), jnp.int32)])
    dst_p = jnp.concatenate([dst, jnp.full((pe,), npad - 1, jnp.int32)])
    et_p = jnp.concatenate([et, jnp.zeros((pe,), jnp.int32)])
    x_pad = jnp.concatenate([ent_embeds, jnp.zeros((npad - n, d), jnp.float32)])

    aggr = _segsum(rel_embeds, et_p, dst_p, npad).reshape(NC, npad, d)
    ones_tab = jnp.ones((8, d), jnp.float32)
    zidx = jnp.zeros((epad,), jnp.int32)
    deg = _segsum(ones_tab, zidx, dst_p, npad, ones_rows=True).reshape(NC, npad, d)

    aggx1 = _segsum(x_pad, src_p, dst_p, npad).reshape(NC, npad, d)
    h1 = _layer(aggx1, aggr, deg, x_pad, W_n1, W_loop1, n, npad)
    aggx2 = _segsum(h1, src_p, dst_p, npad).reshape(NC, npad, d)
    h2 = _layer(aggx2, aggr, deg, h1, W_n2, W_loop2, n, npad)

    nid = node_ids_graph.reshape(-1).astype(jnp.int32)
    emb = _gather(h2, nid).reshape(b, l, d)

    iv = s_time_iterval_sorted.reshape(b * l, 1).astype(jnp.int32)
    lens = jnp.repeat(s_len_non_zero.astype(jnp.int32), l).reshape(b * l, 1)
    pos = jnp.tile(jnp.arange(l, dtype=jnp.int32), b).reshape(b * l, 1)
    te = _time(iv, lens, pos, t_w, t_b).reshape(b, l, -1)

    return jnp.concatenate([emb, te], axis=-1)
